# Initial kernel scaffold; baseline (speedup 1.0000x reference)
import functools

import jax
import jax.numpy as jnp
from jax import lax
from jax.experimental import pallas as pl
from jax.experimental.pallas import tpu as pltpu
from jax.experimental.pallas import tpu_sc as plsc

EMB_DIM = 32
NW = 32
CHUNK = 128
K = 5
GROUP_ROWS = CHUNK * K


def _body(n_chunks, idx_hbm, table_hbm, out_hbm, idx_v, rows_v, gsem):
    c = lax.axis_index("c")
    s = lax.axis_index("s")
    wid = s * 2 + c
    rows_per_w = n_chunks * CHUNK
    base = wid * rows_per_w
    pltpu.sync_copy(idx_hbm.at[wid], idx_v)

    n_groups = n_chunks // K

    def group(go, _):
        cps = []
        for q in range(K):
            cps.append(pltpu.async_copy(
                table_hbm.at[idx_v.at[go * K + q]],
                rows_v.at[pl.ds(q * CHUNK, CHUNK)],
                gsem,
            ))
        for cp in cps:
            cp.wait()
        pltpu.sync_copy(
            rows_v,
            out_hbm.at[pl.ds(base + go * GROUP_ROWS, GROUP_ROWS)],
        )
        return 0

    lax.fori_loop(0, n_groups, group, 0)


def kernel(token_ids, emb_weight):
    orig_shape = token_ids.shape
    b = 1
    for d in orig_shape:
        b *= d
    assert b % (NW * GROUP_ROWS) == 0
    n_chunks = b // (NW * CHUNK)
    idx = token_ids.reshape(NW, n_chunks, CHUNK).astype(jnp.int32)

    mesh = plsc.VectorSubcoreMesh(core_axis_name="c", subcore_axis_name="s")
    gather = pl.kernel(
        functools.partial(_body, n_chunks),
        out_type=jax.ShapeDtypeStruct((b, EMB_DIM), jnp.float32),
        mesh=mesh,
        scratch_types=[
            pltpu.VMEM((n_chunks, CHUNK), jnp.int32),
            pltpu.VMEM((GROUP_ROWS, EMB_DIM), jnp.float32),
            pltpu.SemaphoreType.DMA,
        ],
    )
    out = gather(idx, emb_weight)
    return out.reshape(*orig_shape, EMB_DIM)


# speedup vs baseline: 1.9336x; 1.9336x over previous
import functools

import jax
import jax.numpy as jnp
from jax import lax
from jax.experimental import pallas as pl
from jax.experimental.pallas import tpu as pltpu
from jax.experimental.pallas import tpu_sc as plsc

EMB_DIM = 32
NW = 32
CHUNK = 128
K = 5
GROUP_ROWS = CHUNK * K


def _body(n_chunks, idx_hbm, table_hbm, out_hbm, idx_v, rows_v, gsem):
    c = lax.axis_index("c")
    s = lax.axis_index("s")
    wid = s * 2 + c
    rows_per_w = n_chunks * CHUNK
    base = wid * rows_per_w
    pltpu.sync_copy(idx_hbm.at[wid], idx_v)

    n_groups = n_chunks // K

    def group(go, _):
        cps = []
        for q in range(K):
            cps.append(pltpu.async_copy(
                table_hbm.at[idx_v.at[go * K + q]],
                rows_v.at[pl.ds(q * CHUNK, CHUNK)],
                gsem,
            ))
        for cp in cps:
            cp.wait()
        pltpu.sync_copy(
            rows_v,
            out_hbm.at[pl.ds(base + go * GROUP_ROWS, GROUP_ROWS)],
        )
        return 0

    lax.fori_loop(0, n_groups, group, 0)


def kernel(token_ids, emb_weight):
    orig_shape = token_ids.shape
    b = 1
    for d in orig_shape:
        b *= d
    assert b % (NW * GROUP_ROWS) == 0
    n_chunks = b // (NW * CHUNK)
    idx = token_ids.reshape(NW, n_chunks, CHUNK).astype(jnp.int32)

    mesh = plsc.VectorSubcoreMesh(core_axis_name="c", subcore_axis_name="s")
    gather = pl.kernel(
        functools.partial(_body, n_chunks),
        out_type=jax.ShapeDtypeStruct((b, EMB_DIM), jnp.float32),
        mesh=mesh,
        scratch_types=[
            pltpu.VMEM((n_chunks, CHUNK), jnp.int32),
            pltpu.VMEM((GROUP_ROWS, EMB_DIM), jnp.float32),
            pltpu.SemaphoreType.DMA,
        ],
        compiler_params=pltpu.CompilerParams(use_tc_tiling_on_sc=False),
    )
    out = gather(idx, emb_weight)
    return out.reshape(*orig_shape, EMB_DIM)


# speedup vs baseline: 1.9941x; 1.0313x over previous
import functools

import jax
import jax.numpy as jnp
from jax import lax
from jax.experimental import pallas as pl
from jax.experimental.pallas import tpu as pltpu
from jax.experimental.pallas import tpu_sc as plsc

EMB_DIM = 32
NW = 32
CHUNK = 128
K = 5
GROUP_ROWS = CHUNK * K


def _body(n_chunks, idx_hbm, table_hbm, out_hbm, idx_v, rows_v, gsem, osem):
    c = lax.axis_index("c")
    s = lax.axis_index("s")
    wid = s * 2 + c
    rows_per_w = n_chunks * CHUNK
    base = wid * rows_per_w
    pltpu.sync_copy(idx_hbm.at[wid], idx_v)

    n_groups = n_chunks // K

    def fire_group(g, p):
        for q in range(K):
            pltpu.async_copy(
                table_hbm.at[idx_v.at[g * K + q]],
                rows_v.at[p, pl.ds(q * CHUNK, CHUNK)],
                gsem.at[p],
            )

    def wait_gathers(p):
        pltpu.make_async_copy(
            table_hbm.at[pl.ds(0, GROUP_ROWS)], rows_v.at[p], gsem.at[p]
        ).wait()

    def wait_out(p):
        pltpu.make_async_copy(
            rows_v.at[p], out_hbm.at[pl.ds(0, GROUP_ROWS)], osem.at[p]
        ).wait()

    fire_group(0, 0)

    def body(g, _):
        p = g % 2
        o = 1 - p

        @pl.when(g + 1 < n_groups)
        def _fire_ahead():
            @pl.when(g >= 1)
            def _reclaim():
                wait_out(o)

            fire_group(g + 1, o)

        wait_gathers(p)
        pltpu.async_copy(
            rows_v.at[p],
            out_hbm.at[pl.ds(base + g * GROUP_ROWS, GROUP_ROWS)],
            osem.at[p],
        )
        return 0

    lax.fori_loop(0, n_groups, body, 0)
    wait_out(0)
    wait_out(1)


def kernel(token_ids, emb_weight):
    orig_shape = token_ids.shape
    b = 1
    for d in orig_shape:
        b *= d
    assert b % (NW * GROUP_ROWS) == 0
    n_chunks = b // (NW * CHUNK)
    assert n_chunks // K >= 2
    idx = token_ids.reshape(NW, n_chunks, CHUNK).astype(jnp.int32)

    mesh = plsc.VectorSubcoreMesh(core_axis_name="c", subcore_axis_name="s")
    gather = pl.kernel(
        functools.partial(_body, n_chunks),
        out_type=jax.ShapeDtypeStruct((b, EMB_DIM), jnp.float32),
        mesh=mesh,
        scratch_types=[
            pltpu.VMEM((n_chunks, CHUNK), jnp.int32),
            pltpu.VMEM((2, GROUP_ROWS, EMB_DIM), jnp.float32),
            pltpu.SemaphoreType.DMA((2,)),
            pltpu.SemaphoreType.DMA((2,)),
        ],
        compiler_params=pltpu.CompilerParams(use_tc_tiling_on_sc=False),
    )
    out = gather(idx, emb_weight)
    return out.reshape(*orig_shape, EMB_DIM)
